# baseline (speedup 1.0000x reference)
import functools

import jax
import jax.numpy as jnp
from jax import lax
from jax.experimental import pallas as pl
from jax.experimental.pallas import tpu as pltpu
from jax.experimental.pallas import tpu_sc as plsc

NC = 2
NS = 16
L = 16
NW = NC * NS


def _mf_body(bpw, dim, uids_hbm, iids_hbm, utT_hbm, itT_hbm, out_hbm,
             uidx_v, iidx_v, ug_v, ig_v, out_v, sem_u, sem_i):
    wid = lax.axis_index("s") * NC + lax.axis_index("c")
    base = wid * bpw
    ngroups = bpw // L

    pltpu.sync_copy(uids_hbm.at[pl.ds(base, bpw)], uidx_v)
    pltpu.sync_copy(iids_hbm.at[pl.ds(base, bpw)], iidx_v)

    cps = []
    for k in range(dim):
        cps.append(pltpu.async_copy(utT_hbm.at[k].at[uidx_v],
                                    ug_v.at[k], sem_u))
        cps.append(pltpu.async_copy(itT_hbm.at[k].at[iidx_v],
                                    ig_v.at[k], sem_i))
    for c in cps:
        c.wait()

    def group(g, _):
        s = pl.ds(g * L, L)
        acc = ug_v[0, s] * ig_v[0, s]
        for k in range(1, dim):
            acc = acc + ug_v[k, s] * ig_v[k, s]
        out_v[pl.ds(g * L, L)] = acc
        return 0

    lax.fori_loop(0, ngroups, group, 0)

    pltpu.sync_copy(out_v, out_hbm.at[pl.ds(base, bpw)])


def kernel(uids, iids, user_table, item_table):
    batch = uids.shape[0]
    n, dim = user_table.shape
    bpw = batch // NW

    mesh = plsc.VectorSubcoreMesh(core_axis_name="c", subcore_axis_name="s")
    k = pl.kernel(
        functools.partial(_mf_body, bpw, dim),
        out_type=jax.ShapeDtypeStruct((batch,), jnp.float32),
        mesh=mesh,
        compiler_params=pltpu.CompilerParams(
            needs_layout_passes=False, use_tc_tiling_on_sc=False),
        scratch_types=[
            pltpu.VMEM((bpw,), jnp.int32),
            pltpu.VMEM((bpw,), jnp.int32),
            pltpu.VMEM((dim, bpw), jnp.float32),
            pltpu.VMEM((dim, bpw), jnp.float32),
            pltpu.VMEM((bpw,), jnp.float32),
            pltpu.SemaphoreType.DMA,
            pltpu.SemaphoreType.DMA,
        ],
    )
    return k(uids.astype(jnp.int32), iids.astype(jnp.int32),
             user_table.T, item_table.T)
